# Initial kernel scaffold; baseline (speedup 1.0000x reference)
import jax
import jax.numpy as jnp
from jax.experimental import pallas as pl


def kernel(x, edge_index, W1, b1, W2, b2):
    raise NotImplementedError("write your pallas kernel here")


# speedup vs baseline: 27.4623x; 27.4623x over previous
import functools

import jax
import jax.numpy as jnp
from jax import lax
from jax.experimental import pallas as pl
from jax.experimental.pallas import tpu as pltpu
from jax.experimental.pallas import tpu_sc as plsc

NC = 2
NS = 16
NW = NC * NS
CHUNK = 128


def _sc_mesh():
  return plsc.VectorSubcoreMesh(
      core_axis_name="c", subcore_axis_name="s", num_cores=NC,
      num_subcores=NS)


def _deg_partials(dst3, acc_rows):
  k_chunks = dst3.shape[1]
  zrows = acc_rows // NS
  zpad = ((zrows + 15) // 16) * 16

  @functools.partial(
      pl.kernel,
      out_type=jax.ShapeDtypeStruct((NC * acc_rows,), jnp.float32),
      mesh=_sc_mesh(),
      compiler_params=pltpu.CompilerParams(use_tc_tiling_on_sc=False),
      scratch_types=[
          pltpu.VMEM((k_chunks, CHUNK), jnp.int32),
          pltpu.VMEM((CHUNK,), jnp.float32),
          pltpu.VMEM((zpad,), jnp.float32),
          pltpu.VMEM_SHARED((acc_rows,), jnp.float32),
      ],
  )
  def run(dst_hbm, out_hbm, dst_v, ones_v, z_v, acc_s):
    c = lax.axis_index("c")
    s = lax.axis_index("s")
    w = c * NS + s
    pltpu.sync_copy(dst_hbm.at[w], dst_v)

    @pl.loop(0, CHUNK // 16)
    def _fill(i):
      ones_v[pl.ds(i * 16, 16)] = jnp.ones((16,), jnp.float32)

    @pl.loop(0, zpad // 16)
    def _fillz(i):
      z_v[pl.ds(i * 16, 16)] = jnp.zeros((16,), jnp.float32)

    pltpu.sync_copy(z_v.at[pl.ds(0, zrows)], acc_s.at[pl.ds(s * zrows, zrows)])
    plsc.subcore_barrier()

    @pl.loop(0, k_chunks)
    def _edges(k):
      pltpu.sync_copy(ones_v, acc_s.at[dst_v.at[k]], add=True)

    plsc.subcore_barrier()
    pltpu.sync_copy(acc_s.at[pl.ds(s * zrows, zrows)], z_v.at[pl.ds(0, zrows)])
    pltpu.sync_copy(z_v.at[pl.ds(0, zrows)],
                    out_hbm.at[pl.ds(c * acc_rows + s * zrows, zrows)])

  return run(dst3)


def _prop_partials(gtab, src3, dst3, acc_rows, feat):
  k_chunks = src3.shape[1]
  zrows = acc_rows // NS
  offs = list(range(0, feat - 15, 16))
  if feat % 16:
    offs.append(feat - 16)

  @functools.partial(
      pl.kernel,
      out_type=jax.ShapeDtypeStruct((NC, acc_rows, feat), jnp.float32),
      mesh=_sc_mesh(),
      compiler_params=pltpu.CompilerParams(use_tc_tiling_on_sc=False),
      scratch_types=[
          pltpu.VMEM((k_chunks, CHUNK), jnp.int32),
          pltpu.VMEM((k_chunks, CHUNK), jnp.int32),
          pltpu.VMEM((CHUNK, feat), jnp.float32),
          pltpu.VMEM((zrows, feat), jnp.float32),
          pltpu.VMEM_SHARED((acc_rows, feat), jnp.float32),
          pltpu.SemaphoreType.DMA,
      ],
  )
  def run(g_hbm, src_hbm, dst_hbm, out_hbm,
          src_v, dst_v, rows_v, z_v, acc_s, sem):
    c = lax.axis_index("c")
    s = lax.axis_index("s")
    w = c * NS + s
    pltpu.sync_copy(src_hbm.at[w], src_v)
    pltpu.sync_copy(dst_hbm.at[w], dst_v)

    @pl.loop(0, zrows)
    def _fillz(r):
      for o in offs:
        z_v[r, pl.ds(o, 16)] = jnp.zeros((16,), jnp.float32)

    pltpu.sync_copy(z_v, acc_s.at[pl.ds(s * zrows, zrows)])
    plsc.subcore_barrier()

    @pl.loop(0, k_chunks)
    def _edges(k):
      pltpu.async_copy(g_hbm.at[src_v.at[k]], rows_v, sem).wait()
      pltpu.sync_copy(rows_v, acc_s.at[dst_v.at[k]], add=True)

    plsc.subcore_barrier()
    pltpu.sync_copy(acc_s.at[pl.ds(s * zrows, zrows)], z_v)
    pltpu.sync_copy(z_v, out_hbm.at[c, pl.ds(s * zrows, zrows)])

  return run(gtab, src3, dst3)


def _tc_first(deg_p, x, w1, bm):
  n, d_in = x.shape
  hid = w1.shape[1]
  grid = n // bm

  def body(deg_ref, x_ref, w1_ref, dis_ref, g1_ref):
    deg = deg_ref[0] + deg_ref[1] + 1.0
    dis = lax.rsqrt(deg)
    h1 = jnp.dot(x_ref[...], w1_ref[...], preferred_element_type=jnp.float32)
    dis_ref[...] = dis
    g1_ref[...] = dis * h1

  return pl.pallas_call(
      body,
      grid=(grid,),
      in_specs=[
          pl.BlockSpec((NC, bm, 1), lambda i: (0, i, 0)),
          pl.BlockSpec((bm, d_in), lambda i: (i, 0)),
          pl.BlockSpec((d_in, hid), lambda i: (0, 0)),
      ],
      out_specs=[
          pl.BlockSpec((bm, 1), lambda i: (i, 0)),
          pl.BlockSpec((bm, hid), lambda i: (i, 0)),
      ],
      out_shape=[
          jax.ShapeDtypeStruct((n, 1), jnp.float32),
          jax.ShapeDtypeStruct((n, hid), jnp.float32),
      ],
  )(deg_p, x, w1)


def _tc_mid(acc_p, g1, dis, b1, w2, bm):
  n, hid = g1.shape
  ncls = w2.shape[1]
  grid = n // bm

  def body(acc_ref, g1_ref, dis_ref, b1_ref, w2_ref, g2_ref):
    a = acc_ref[0] + acc_ref[1] + g1_ref[...]
    z = jnp.maximum(dis_ref[...] * a + b1_ref[...], 0.0)
    h2 = jnp.dot(z, w2_ref[...], preferred_element_type=jnp.float32)
    g2_ref[...] = dis_ref[...] * h2

  return pl.pallas_call(
      body,
      grid=(grid,),
      in_specs=[
          pl.BlockSpec((NC, bm, hid), lambda i: (0, i, 0)),
          pl.BlockSpec((bm, hid), lambda i: (i, 0)),
          pl.BlockSpec((bm, 1), lambda i: (i, 0)),
          pl.BlockSpec((1, hid), lambda i: (0, 0)),
          pl.BlockSpec((hid, ncls), lambda i: (0, 0)),
      ],
      out_specs=pl.BlockSpec((bm, ncls), lambda i: (i, 0)),
      out_shape=jax.ShapeDtypeStruct((n, ncls), jnp.float32),
  )(acc_p, g1, dis, b1, w2)


def _tc_last(acc_p, g2, dis, b2, bm):
  n, ncls = g2.shape
  grid = n // bm

  def body(acc_ref, g2_ref, dis_ref, b2_ref, out_ref):
    a = acc_ref[0] + acc_ref[1] + g2_ref[...]
    out_ref[...] = dis_ref[...] * a + b2_ref[...]

  return pl.pallas_call(
      body,
      grid=(grid,),
      in_specs=[
          pl.BlockSpec((NC, bm, ncls), lambda i: (0, i, 0)),
          pl.BlockSpec((bm, ncls), lambda i: (i, 0)),
          pl.BlockSpec((bm, 1), lambda i: (i, 0)),
          pl.BlockSpec((1, ncls), lambda i: (0, 0)),
      ],
      out_specs=pl.BlockSpec((bm, ncls), lambda i: (i, 0)),
      out_shape=jax.ShapeDtypeStruct((n, ncls), jnp.float32),
  )(acc_p, g2, dis, b2)


def kernel(x, edge_index, W1, b1, W2, b2):
  n, _ = x.shape
  hid = W1.shape[1]
  ncls = W2.shape[1]
  e = edge_index.shape[1]

  acc_rows = ((n + 1 + 127) // 128) * 128
  zrows = acc_rows // NS
  bm = 2000

  ew = NW * CHUNK
  k_chunks = (e + ew - 1) // ew
  e_pad = k_chunks * ew
  src = edge_index[0]
  dst = edge_index[1]
  pad = e_pad - e
  src3 = jnp.concatenate(
      [src, jnp.zeros((pad,), jnp.int32)]).reshape(NW, k_chunks, CHUNK)
  dst3 = jnp.concatenate(
      [dst, jnp.full((pad,), n, jnp.int32)]).reshape(NW, k_chunks, CHUNK)

  deg_p = _deg_partials(dst3, acc_rows)
  dis, g1 = _tc_first(deg_p.reshape(NC, acc_rows, 1), x, W1, bm)
  acc1 = _prop_partials(g1, src3, dst3, acc_rows, hid)
  g2 = _tc_mid(acc1, g1, dis, b1.reshape(1, hid), W2, bm)
  acc2 = _prop_partials(g2, src3, dst3, acc_rows, ncls)
  return _tc_last(acc2, g2, dis, b2.reshape(1, ncls), bm)


# speedup vs baseline: 28.7456x; 1.0467x over previous
import functools

import jax
import jax.numpy as jnp
from jax import lax
from jax.experimental import pallas as pl
from jax.experimental.pallas import tpu as pltpu
from jax.experimental.pallas import tpu_sc as plsc

NC = 2
NS = 16
NW = NC * NS
CHUNK = 128
NBUF = 4


def _sc_mesh():
  return plsc.VectorSubcoreMesh(
      core_axis_name="c", subcore_axis_name="s", num_cores=NC,
      num_subcores=NS)


def _deg_partials(dst3, acc_rows):
  k_chunks = dst3.shape[1]
  zrows = acc_rows // NS
  zpad = ((zrows + 15) // 16) * 16

  @functools.partial(
      pl.kernel,
      out_type=jax.ShapeDtypeStruct((NC * acc_rows,), jnp.float32),
      mesh=_sc_mesh(),
      compiler_params=pltpu.CompilerParams(use_tc_tiling_on_sc=False),
      scratch_types=[
          pltpu.VMEM((k_chunks, CHUNK), jnp.int32),
          pltpu.VMEM((CHUNK,), jnp.float32),
          pltpu.VMEM((zpad,), jnp.float32),
          pltpu.VMEM_SHARED((acc_rows,), jnp.float32),
      ],
  )
  def run(dst_hbm, out_hbm, dst_v, ones_v, z_v, acc_s):
    c = lax.axis_index("c")
    s = lax.axis_index("s")
    w = c * NS + s
    pltpu.sync_copy(dst_hbm.at[w], dst_v)

    @pl.loop(0, CHUNK // 16)
    def _fill(i):
      ones_v[pl.ds(i * 16, 16)] = jnp.ones((16,), jnp.float32)

    @pl.loop(0, zpad // 16)
    def _fillz(i):
      z_v[pl.ds(i * 16, 16)] = jnp.zeros((16,), jnp.float32)

    pltpu.sync_copy(z_v.at[pl.ds(0, zrows)], acc_s.at[pl.ds(s * zrows, zrows)])
    plsc.subcore_barrier()

    @pl.loop(0, k_chunks)
    def _edges(k):
      pltpu.sync_copy(ones_v, acc_s.at[dst_v.at[k]], add=True)

    plsc.subcore_barrier()
    pltpu.sync_copy(acc_s.at[pl.ds(s * zrows, zrows)], z_v.at[pl.ds(0, zrows)])
    pltpu.sync_copy(z_v.at[pl.ds(0, zrows)],
                    out_hbm.at[pl.ds(c * acc_rows + s * zrows, zrows)])

  return run(dst3)


def _prop_partials(gtab, src3, dst3, acc_rows, feat):
  k_chunks = src3.shape[1]
  zrows = acc_rows // NS
  offs = list(range(0, feat - 15, 16))
  if feat % 16:
    offs.append(feat - 16)

  @functools.partial(
      pl.kernel,
      out_type=jax.ShapeDtypeStruct((NC, acc_rows, feat), jnp.float32),
      mesh=_sc_mesh(),
      compiler_params=pltpu.CompilerParams(use_tc_tiling_on_sc=False),
      scratch_types=[
          pltpu.VMEM((k_chunks, CHUNK), jnp.int32),
          pltpu.VMEM((k_chunks, CHUNK), jnp.int32),
          pltpu.VMEM((NBUF, CHUNK, feat), jnp.float32),
          pltpu.VMEM((zrows, feat), jnp.float32),
          pltpu.VMEM_SHARED((acc_rows, feat), jnp.float32),
      ] + [pltpu.SemaphoreType.DMA] * NBUF,
  )
  def run(g_hbm, src_hbm, dst_hbm, out_hbm,
          src_v, dst_v, rows_v, z_v, acc_s, *sems):
    c = lax.axis_index("c")
    s = lax.axis_index("s")
    w = c * NS + s
    pltpu.sync_copy(src_hbm.at[w], src_v)
    pltpu.sync_copy(dst_hbm.at[w], dst_v)

    @pl.loop(0, zrows)
    def _fillz(r):
      for o in offs:
        z_v[r, pl.ds(o, 16)] = jnp.zeros((16,), jnp.float32)

    pltpu.sync_copy(z_v, acc_s.at[pl.ds(s * zrows, zrows)])
    plsc.subcore_barrier()

    for b in range(NBUF):
      pltpu.async_copy(g_hbm.at[src_v.at[b]], rows_v.at[b], sems[b])

    @pl.loop(0, k_chunks // NBUF)
    def _edges(g):
      for b in range(NBUF):
        k = g * NBUF + b
        pltpu.make_async_copy(g_hbm.at[src_v.at[k]], rows_v.at[b],
                              sems[b]).wait()
        pltpu.sync_copy(rows_v.at[b], acc_s.at[dst_v.at[k]], add=True)
        nxt = k + NBUF

        @pl.when(nxt < k_chunks)
        def _refire():
          pltpu.async_copy(g_hbm.at[src_v.at[nxt]], rows_v.at[b], sems[b])

    plsc.subcore_barrier()
    pltpu.sync_copy(acc_s.at[pl.ds(s * zrows, zrows)], z_v)
    pltpu.sync_copy(z_v, out_hbm.at[c, pl.ds(s * zrows, zrows)])

  return run(gtab, src3, dst3)


def _tc_first(deg_p, x, w1, bm):
  n, d_in = x.shape
  hid = w1.shape[1]
  grid = n // bm

  def body(deg_ref, x_ref, w1_ref, dis_ref, g1_ref):
    deg = deg_ref[0] + deg_ref[1] + 1.0
    dis = lax.rsqrt(deg)
    h1 = jnp.dot(x_ref[...], w1_ref[...], preferred_element_type=jnp.float32)
    dis_ref[...] = dis
    g1_ref[...] = dis * h1

  return pl.pallas_call(
      body,
      grid=(grid,),
      in_specs=[
          pl.BlockSpec((NC, bm, 1), lambda i: (0, i, 0)),
          pl.BlockSpec((bm, d_in), lambda i: (i, 0)),
          pl.BlockSpec((d_in, hid), lambda i: (0, 0)),
      ],
      out_specs=[
          pl.BlockSpec((bm, 1), lambda i: (i, 0)),
          pl.BlockSpec((bm, hid), lambda i: (i, 0)),
      ],
      out_shape=[
          jax.ShapeDtypeStruct((n, 1), jnp.float32),
          jax.ShapeDtypeStruct((n, hid), jnp.float32),
      ],
  )(deg_p, x, w1)


def _tc_mid(acc_p, g1, dis, b1, w2, bm):
  n, hid = g1.shape
  ncls = w2.shape[1]
  grid = n // bm

  def body(acc_ref, g1_ref, dis_ref, b1_ref, w2_ref, g2_ref):
    a = acc_ref[0] + acc_ref[1] + g1_ref[...]
    z = jnp.maximum(dis_ref[...] * a + b1_ref[...], 0.0)
    h2 = jnp.dot(z, w2_ref[...], preferred_element_type=jnp.float32)
    g2_ref[...] = dis_ref[...] * h2

  return pl.pallas_call(
      body,
      grid=(grid,),
      in_specs=[
          pl.BlockSpec((NC, bm, hid), lambda i: (0, i, 0)),
          pl.BlockSpec((bm, hid), lambda i: (i, 0)),
          pl.BlockSpec((bm, 1), lambda i: (i, 0)),
          pl.BlockSpec((1, hid), lambda i: (0, 0)),
          pl.BlockSpec((hid, ncls), lambda i: (0, 0)),
      ],
      out_specs=pl.BlockSpec((bm, ncls), lambda i: (i, 0)),
      out_shape=jax.ShapeDtypeStruct((n, ncls), jnp.float32),
  )(acc_p, g1, dis, b1, w2)


def _tc_last(acc_p, g2, dis, b2, bm):
  n, ncls = g2.shape
  grid = n // bm

  def body(acc_ref, g2_ref, dis_ref, b2_ref, out_ref):
    a = acc_ref[0] + acc_ref[1] + g2_ref[...]
    out_ref[...] = dis_ref[...] * a + b2_ref[...]

  return pl.pallas_call(
      body,
      grid=(grid,),
      in_specs=[
          pl.BlockSpec((NC, bm, ncls), lambda i: (0, i, 0)),
          pl.BlockSpec((bm, ncls), lambda i: (i, 0)),
          pl.BlockSpec((bm, 1), lambda i: (i, 0)),
          pl.BlockSpec((1, ncls), lambda i: (0, 0)),
      ],
      out_specs=pl.BlockSpec((bm, ncls), lambda i: (i, 0)),
      out_shape=jax.ShapeDtypeStruct((n, ncls), jnp.float32),
  )(acc_p, g2, dis, b2)


def kernel(x, edge_index, W1, b1, W2, b2):
  n, _ = x.shape
  hid = W1.shape[1]
  ncls = W2.shape[1]
  e = edge_index.shape[1]

  acc_rows = ((n + 1 + 127) // 128) * 128
  zrows = acc_rows // NS
  bm = 2000

  ew = NW * CHUNK
  k_chunks = ((e + ew - 1) // ew + NBUF - 1) // NBUF * NBUF
  e_pad = k_chunks * ew
  src = edge_index[0]
  dst = edge_index[1]
  pad = e_pad - e
  src3 = jnp.concatenate(
      [src, jnp.zeros((pad,), jnp.int32)]).reshape(NW, k_chunks, CHUNK)
  dst3 = jnp.concatenate(
      [dst, jnp.full((pad,), n, jnp.int32)]).reshape(NW, k_chunks, CHUNK)

  deg_p = _deg_partials(dst3, acc_rows)
  dis, g1 = _tc_first(deg_p.reshape(NC, acc_rows, 1), x, W1, bm)
  acc1 = _prop_partials(g1, src3, dst3, acc_rows, hid)
  g2 = _tc_mid(acc1, g1, dis, b1.reshape(1, hid), W2, bm)
  acc2 = _prop_partials(g2, src3, dst3, acc_rows, ncls)
  return _tc_last(acc2, g2, dis, b2.reshape(1, ncls), bm)
